# baseline (speedup 1.0000x reference)
import functools
import math

import jax
import jax.numpy as jnp
from jax import lax
from jax.experimental import pallas as pl
from jax.experimental.pallas import tpu as pltpu
from jax.experimental.pallas import tpu_sc as plsc

D = 128
PAD = 5000
SCALE = math.sqrt(float(D))
NC, NS, L = 2, 16, 16
NW = NC * NS
C = 128
PE_ROWS = 5008
DW = D // 2


@functools.lru_cache(maxsize=None)
def _build(N):
    rows_per_w = N // NW
    n_chunks = rows_per_w // C
    assert rows_per_w % C == 0 and n_chunks >= 4 and n_chunks % 2 == 0
    mesh = plsc.VectorSubcoreMesh(core_axis_name="c", subcore_axis_name="s")

    @functools.partial(
        pl.kernel,
        out_type=jax.ShapeDtypeStruct((N, D), jnp.float32),
        mesh=mesh,
        compiler_params=pltpu.CompilerParams(use_tc_tiling_on_sc=False),
        scratch_types=[
            pltpu.VMEM((n_chunks, C), jnp.int32),
            pltpu.VMEM((n_chunks, C), jnp.int32),
            [pltpu.VMEM((C, D), jnp.float32)] * 2,
            [pltpu.VMEM((C, DW), jnp.int32)] * 2,
            [pltpu.VMEM((C, D), jnp.float32)] * 2,
            [pltpu.SemaphoreType.DMA] * 2,
            [pltpu.SemaphoreType.DMA] * 2,
            [pltpu.SemaphoreType.DMA] * 2,
            pltpu.VMEM_SHARED((PE_ROWS, DW), jnp.int32),
        ],
    )
    def k(x_hbm, msk_hbm, idx_hbm, pe_hbm, out_hbm,
          idx_v, msk_v, x_v, rows_v, out_v, sem_x, sem_g, sem_o, pe_sh):
        wid = lax.axis_index("s") * NC + lax.axis_index("c")
        base = wid * rows_per_w

        @pl.when(lax.axis_index("s") == 0)
        def _stage():
            pltpu.sync_copy(pe_hbm, pe_sh)

        pltpu.sync_copy(idx_hbm.at[wid], idx_v)
        pltpu.sync_copy(msk_hbm.at[wid], msk_v)

        @plsc.parallel_loop(0, n_chunks, unroll=2)
        def _fix(r):
            for cb in range(C // L):
                s = pl.ds(cb * L, L)
                iv = jnp.minimum(idx_v[r, s], PAD)
                idx_v[r, s] = jnp.where(msk_v[r, s] != 0, PAD, iv)

        plsc.subcore_barrier()

        def in_copies(g, b):
            gat = pltpu.make_async_copy(pe_sh.at[idx_v.at[g]], rows_v[b], sem_g[b])
            xcp = pltpu.make_async_copy(x_hbm.at[pl.ds(base + g * C, C)], x_v[b], sem_x[b])
            return gat, xcp

        def out_copy(g, b):
            return pltpu.make_async_copy(out_v[b], out_hbm.at[pl.ds(base + g * C, C)], sem_o[b])

        def start_in(g, b):
            gat, xcp = in_copies(g, b)
            gat.start()
            xcp.start()

        def wait_in(g, b):
            gat, xcp = in_copies(g, b)
            gat.wait()
            xcp.wait()

        def fma(b):
            xb, rb, ob = x_v[b], rows_v[b], out_v[b]

            @plsc.parallel_loop(0, C, unroll=2)
            def _fma(r):
                for kblk in range(D // (2 * L)):
                    w = rb[r, pl.ds(kblk * L, L)]
                    pa = lax.bitcast_convert_type(w << 16, jnp.float32)
                    pb = lax.bitcast_convert_type(w & jnp.int32(-65536), jnp.float32)
                    sa = pl.ds(kblk * 2 * L, L)
                    sb = pl.ds(kblk * 2 * L + L, L)
                    ob[r, sa] = SCALE * xb[r, sa] + pa
                    ob[r, sb] = SCALE * xb[r, sb] + pb

        for b in range(2):
            start_in(b, b)

        for g in range(2):
            wait_in(g, g)
            fma(g)
            out_copy(g, g).start()
            start_in(g + 2, g)

        def body(kk, carry):
            for j in range(2):
                g = 2 * kk + j
                wait_in(g, j)
                out_copy(g - 2, j).wait()
                fma(j)
                out_copy(g, j).start()
                start_in(g + 2, j)
            return carry

        lax.fori_loop(1, n_chunks // 2 - 1, body, 0)

        for g in range(n_chunks - 2, n_chunks):
            b = g % 2
            wait_in(g, b)
            out_copy(g - 2, b).wait()
            fma(b)
            out_copy(g, b).start()
        for g in range(n_chunks - 2, n_chunks):
            out_copy(g, g % 2).wait()

    return k


def kernel(x, mask, indices, pe):
    B, S, Dm = x.shape
    N = B * S
    x2 = x.reshape(N, Dm)
    n_chunks = N // (NW * C)
    msk = mask.reshape(NW, n_chunks, C).astype(jnp.int32)
    idx = indices.reshape(NW, n_chunks, C).astype(jnp.int32)
    pe_p = jnp.pad(pe, ((0, PE_ROWS - pe.shape[0]), (0, 0)))
    pe_r = pe_p.astype(jnp.bfloat16).reshape(PE_ROWS, D // (2 * L), 2, L)
    pe_i = pe_r.transpose(0, 1, 3, 2).reshape(PE_ROWS, DW, 2)
    pe_w = lax.bitcast_convert_type(pe_i, jnp.int32)
    out = _build(N)(x2, msk, idx, pe_w)
    return out.reshape(B, S, Dm)


# speedup vs baseline: 1.0396x; 1.0396x over previous
import functools
import math

import jax
import jax.numpy as jnp
from jax import lax
from jax.experimental import pallas as pl
from jax.experimental.pallas import tpu as pltpu
from jax.experimental.pallas import tpu_sc as plsc

D = 128
PAD = 5000
SCALE = math.sqrt(float(D))
NC, NS, L = 2, 16, 16
NW = NC * NS
C = 64
PE_ROWS = 5008
DW = D // 2


@functools.lru_cache(maxsize=None)
def _build(N):
    rows_per_w = N // NW
    n_chunks = rows_per_w // C
    assert rows_per_w % C == 0 and n_chunks >= 12 and n_chunks % 4 == 0
    mesh = plsc.VectorSubcoreMesh(core_axis_name="c", subcore_axis_name="s")

    @functools.partial(
        pl.kernel,
        out_type=jax.ShapeDtypeStruct((N, D), jnp.float32),
        mesh=mesh,
        compiler_params=pltpu.CompilerParams(use_tc_tiling_on_sc=False),
        scratch_types=[
            pltpu.VMEM((n_chunks, C), jnp.int32),
            pltpu.VMEM((n_chunks, C), jnp.int32),
            [pltpu.VMEM((C, D), jnp.float32)] * 4,
            [pltpu.VMEM((C, DW), jnp.int32)] * 4,
            [pltpu.VMEM((C, D), jnp.float32)] * 4,
            [pltpu.SemaphoreType.DMA] * 4,
            [pltpu.SemaphoreType.DMA] * 4,
            [pltpu.SemaphoreType.DMA] * 4,
            pltpu.VMEM_SHARED((PE_ROWS, DW), jnp.int32),
        ],
    )
    def k(x_hbm, msk_hbm, idx_hbm, pe_hbm, out_hbm,
          idx_v, msk_v, x_v, rows_v, out_v, sem_x, sem_g, sem_o, pe_sh):
        wid = lax.axis_index("s") * NC + lax.axis_index("c")
        base = wid * rows_per_w

        @pl.when(lax.axis_index("s") == 0)
        def _stage():
            pltpu.sync_copy(pe_hbm, pe_sh)

        pltpu.sync_copy(idx_hbm.at[wid], idx_v)
        pltpu.sync_copy(msk_hbm.at[wid], msk_v)

        @plsc.parallel_loop(0, n_chunks, unroll=2)
        def _fix(r):
            for cb in range(C // L):
                s = pl.ds(cb * L, L)
                iv = jnp.minimum(idx_v[r, s], PAD)
                idx_v[r, s] = jnp.where(msk_v[r, s] != 0, PAD, iv)

        plsc.subcore_barrier()

        def in_copies(g, b):
            gat = pltpu.make_async_copy(pe_sh.at[idx_v.at[g]], rows_v[b], sem_g[b])
            xcp = pltpu.make_async_copy(x_hbm.at[pl.ds(base + g * C, C)], x_v[b], sem_x[b])
            return gat, xcp

        def out_copy(g, b):
            return pltpu.make_async_copy(out_v[b], out_hbm.at[pl.ds(base + g * C, C)], sem_o[b])

        def start_in(g, b):
            gat, xcp = in_copies(g, b)
            gat.start()
            xcp.start()

        def wait_in(g, b):
            gat, xcp = in_copies(g, b)
            gat.wait()
            xcp.wait()

        def fma(b):
            xb, rb, ob = x_v[b], rows_v[b], out_v[b]

            @plsc.parallel_loop(0, C, unroll=2)
            def _fma(r):
                for kblk in range(D // (2 * L)):
                    w = rb[r, pl.ds(kblk * L, L)]
                    pa = lax.bitcast_convert_type(w << 16, jnp.float32)
                    pb = lax.bitcast_convert_type(w & jnp.int32(-65536), jnp.float32)
                    sa = pl.ds(kblk * 2 * L, L)
                    sb = pl.ds(kblk * 2 * L + L, L)
                    ob[r, sa] = SCALE * xb[r, sa] + pa
                    ob[r, sb] = SCALE * xb[r, sb] + pb

        for b in range(4):
            start_in(b, b)

        for g in range(4):
            wait_in(g, g)
            fma(g)
            out_copy(g, g).start()
            start_in(g + 4, g)

        def body(kk, carry):
            for j in range(4):
                g = 4 * kk + j
                wait_in(g, j)
                out_copy(g - 4, j).wait()
                fma(j)
                out_copy(g, j).start()
                start_in(g + 4, j)
            return carry

        lax.fori_loop(1, n_chunks // 4 - 1, body, 0)

        for g in range(n_chunks - 4, n_chunks):
            b = g % 4
            wait_in(g, b)
            out_copy(g - 4, b).wait()
            fma(b)
            out_copy(g, b).start()
        for g in range(n_chunks - 4, n_chunks):
            out_copy(g, g % 4).wait()

    return k


def kernel(x, mask, indices, pe):
    B, S, Dm = x.shape
    N = B * S
    x2 = x.reshape(N, Dm)
    n_chunks = N // (NW * C)
    msk = mask.reshape(NW, n_chunks, C).astype(jnp.int32)
    idx = indices.reshape(NW, n_chunks, C).astype(jnp.int32)
    pe_p = jnp.pad(pe, ((0, PE_ROWS - pe.shape[0]), (0, 0)))
    pe_r = pe_p.astype(jnp.bfloat16).reshape(PE_ROWS, D // (2 * L), 2, L)
    pe_i = pe_r.transpose(0, 1, 3, 2).reshape(PE_ROWS, DW, 2)
    pe_w = lax.bitcast_convert_type(pe_i, jnp.int32)
    out = _build(N)(x2, msk, idx, pe_w)
    return out.reshape(B, S, Dm)
